# baseline (speedup 1.0000x reference)
import functools

import jax
import jax.numpy as jnp
from jax import lax
from jax.experimental import pallas as pl
from jax.experimental.pallas import tpu as pltpu
from jax.experimental.pallas import tpu_sc as plsc

SCALE = 8.0
NBUF = 4


@functools.cache
def _build(B0, S, V, D):
    info = plsc.get_sparse_core_info()
    NC, NS, L = info.num_cores, info.num_subcores, info.num_lanes
    NW = NC * NS
    assert B0 % (NW * NBUF) == 0
    rows_per_w = B0 // NW
    n_grp = rows_per_w // NBUF

    mesh = plsc.VectorSubcoreMesh(core_axis_name="c", subcore_axis_name="s")

    scratch = (
        [pltpu.VMEM((rows_per_w, S), jnp.int32)]
        + [pltpu.VMEM((S, D), jnp.float32) for _ in range(2 * NBUF)]
        + [pltpu.SemaphoreType.DMA for _ in range(2 * NBUF)]
    )

    @functools.partial(
        pl.kernel,
        mesh=mesh,
        out_type=jax.ShapeDtypeStruct((B0, S, D), jnp.float32),
        scratch_types=scratch,
        compiler_params=pltpu.CompilerParams(use_tc_tiling_on_sc=False),
    )
    def emb(idx_hbm, table_hbm, out_hbm, idx_v, *bufs):
        rows_g = list(bufs[:NBUF])
        rows_o = list(bufs[NBUF : 2 * NBUF])
        gsem = list(bufs[2 * NBUF : 3 * NBUF])
        osem = list(bufs[3 * NBUF : 4 * NBUF])

        wid = lax.axis_index("s") * NC + lax.axis_index("c")
        row0 = wid * rows_per_w

        pltpu.sync_copy(idx_hbm.at[pl.ds(row0, rows_per_w)], idx_v)

        def start_gather(b, q):
            pltpu.async_copy(table_hbm.at[idx_v.at[q]], rows_g[b], gsem[b])

        for b in range(NBUF):
            start_gather(b, b)

        def grp_body(gi, carry):
            for b in range(NBUF):
                q = gi * NBUF + b
                pltpu.make_async_copy(
                    table_hbm.at[idx_v.at[0]], rows_g[b], gsem[b]
                ).wait()
                @pl.when(gi >= 1)
                def _():
                    pltpu.make_async_copy(
                        rows_o[b], out_hbm.at[row0 + q], osem[b]
                    ).wait()

                def scale_body(r4, c):
                    for rr in range(4):
                        r = r4 * 4 + rr
                        for j in range(D // L):
                            sl = pl.ds(j * L, L)
                            rows_o[b][r, sl] = rows_g[b][r, sl] * SCALE
                    return c

                lax.fori_loop(0, S // 4, scale_body, 0)
                pltpu.async_copy(rows_o[b], out_hbm.at[row0 + q], osem[b])

                @pl.when(gi < n_grp - 1)
                def _():
                    start_gather(b, q + NBUF)

            return carry

        lax.fori_loop(0, n_grp, grp_body, 0)

        for b in range(NBUF):
            q = rows_per_w - NBUF + b
            pltpu.make_async_copy(rows_o[b], out_hbm.at[row0 + q], osem[b]).wait()

    return emb


def kernel(inputs, table):
    B0, S = inputs.shape
    V, D = table.shape
    return _build(B0, S, V, D)(inputs.astype(jnp.int32), table)


# speedup vs baseline: 1.1054x; 1.1054x over previous
import functools

import jax
import jax.numpy as jnp
from jax import lax
from jax.experimental import pallas as pl
from jax.experimental.pallas import tpu as pltpu
from jax.experimental.pallas import tpu_sc as plsc

SCALE = 8.0
C = 100


@functools.cache
def _build(B0, S, V, D):
    info = plsc.get_sparse_core_info()
    NC, NS, L = info.num_cores, info.num_subcores, info.num_lanes
    NW = NC * NS
    assert S == 2 * C and B0 % (2 * NW) == 0
    rows_per_w = B0 // NW
    n_chunks = 2 * rows_per_w
    n_grp = rows_per_w // 2

    mesh = plsc.VectorSubcoreMesh(core_axis_name="c", subcore_axis_name="s")

    scratch = (
        [pltpu.VMEM((n_chunks, C), jnp.int32)]
        + [pltpu.VMEM((C, 2 * D), jnp.float32) for _ in range(2)]
        + [pltpu.VMEM((S, D), jnp.float32) for _ in range(2)]
        + [pltpu.SemaphoreType.DMA for _ in range(4)]
    )

    @functools.partial(
        pl.kernel,
        mesh=mesh,
        out_type=jax.ShapeDtypeStruct((B0, S, D), jnp.float32),
        scratch_types=scratch,
    )
    def emb(idx_hbm, table_hbm, out_hbm, idx_v, g0, g1, o0, o1, gs0, gs1, os0, os1):
        rows_g = [g0, g1]
        rows_o = [o0, o1]
        gsem = [gs0, gs1]
        osem = [os0, os1]

        wid = lax.axis_index("s") * NC + lax.axis_index("c")
        crow0 = wid * n_chunks
        orow0 = wid * rows_per_w

        pltpu.sync_copy(idx_hbm.at[pl.ds(crow0, n_chunks)], idx_v)

        def start_gather(h, c):
            pltpu.async_copy(table_hbm.at[idx_v.at[c]], rows_g[h], gsem[h])

        start_gather(0, 0)
        start_gather(1, 1)

        def grp_body(gi, carry):
            for lq in range(2):
                q = gi * 2 + lq
                @pl.when(gi >= 1)
                def _():
                    pltpu.make_async_copy(
                        rows_o[lq], out_hbm.at[orow0 + q], osem[lq]
                    ).wait()

                for h in range(2):
                    pltpu.make_async_copy(
                        table_hbm.at[idx_v.at[0]], rows_g[h], gsem[h]
                    ).wait()

                    def scale_body(r4, c, h=h, lq=lq):
                        for rr in range(4):
                            r = r4 * 4 + rr
                            for j in range(D // L):
                                rows_o[lq][h * C + r, pl.ds(j * L, L)] = (
                                    rows_g[h][r, pl.ds(j * L, L)] * SCALE
                                )
                        return c

                    lax.fori_loop(0, C // 4, scale_body, 0)

                    @pl.when(q < rows_per_w - 1)
                    def _():
                        start_gather(h, (q + 1) * 2 + h)

                pltpu.async_copy(rows_o[lq], out_hbm.at[orow0 + q], osem[lq])

            return carry

        lax.fori_loop(0, n_grp, grp_body, 0)

        for lq in range(2):
            q = rows_per_w - 2 + lq
            pltpu.make_async_copy(
                rows_o[lq], out_hbm.at[orow0 + q], osem[lq]
            ).wait()

    return emb


def kernel(inputs, table):
    B0, S = inputs.shape
    V, D = table.shape
    idx = inputs.reshape(B0 * S // C, C).astype(jnp.int32)
    t128 = jnp.pad(table, ((0, 0), (0, D)))
    return _build(B0, S, V, D)(idx, t128)
